# baseline (speedup 1.0000x reference)
import functools

import jax
import jax.numpy as jnp
from jax import lax
from jax.experimental import pallas as pl
from jax.experimental.pallas import tpu as pltpu
from jax.experimental.pallas import tpu_sc as plsc

N = 10000
D = 256
C = 64
HALF = 128
CHUNK = 128
E_PAD = 163840
ROWS = E_PAD // CHUNK
N_TILES = 16
ROWS_L1 = ROWS // N_TILES
ROWS_L2 = ROWS // 2 // N_TILES
ROWS_L2A = 16
ROWS_L2B = 64
SPLIT_L2 = ROWS_L2A * N_TILES
NPAD = 10240
NPT = NPAD // N_TILES
ZROWS = 128
BLK = 1000


def _clip_body(x_ref, lo_ref, hi_ref):
    xb = x_ref[...]
    n2 = jnp.sum(xb * xb, axis=1, keepdims=True)
    xc = xb * (1.0 / jnp.maximum(jnp.sqrt(n2), 1.0))
    lo_ref[...] = xc[:, :HALF]
    hi_ref[...] = xc[:, HALF:]


_clip = pl.pallas_call(
    _clip_body,
    grid=(N // BLK,),
    in_specs=[pl.BlockSpec((BLK, D), lambda i: (i, 0))],
    out_specs=[pl.BlockSpec((BLK, HALF), lambda i: (i, 0))] * 2,
    out_shape=[jax.ShapeDtypeStruct((N, HALF), jnp.float32)] * 2,
)


def _mid_body(lo_ref, hi_ref, slo_ref, shi_ref, w1_ref, b1_ref, w2_ref, z_ref):
    alo = lo_ref[...] + slo_ref[...]
    ahi = hi_ref[...] + shi_ref[...]
    w1 = w1_ref[...]
    h = jnp.dot(alo, w1[:HALF, :], preferred_element_type=jnp.float32)
    h = h + jnp.dot(ahi, w1[HALF:, :], preferred_element_type=jnp.float32)
    h = jnp.maximum(h + b1_ref[...], 0.0)
    n2 = jnp.sum(h * h, axis=1, keepdims=True)
    hc = h * (1.0 / jnp.maximum(jnp.sqrt(n2), 1.0))
    z = jnp.dot(hc, w2_ref[...], preferred_element_type=jnp.float32)
    z_ref[...] = jnp.concatenate([z, jnp.zeros_like(z)], axis=1)


_mid = pl.pallas_call(
    _mid_body,
    grid=(N // BLK,),
    in_specs=[
        pl.BlockSpec((BLK, HALF), lambda i: (i, 0)),
        pl.BlockSpec((BLK, HALF), lambda i: (i, 0)),
        pl.BlockSpec((BLK, HALF), lambda i: (i, 0)),
        pl.BlockSpec((BLK, HALF), lambda i: (i, 0)),
        pl.BlockSpec((D, D), lambda i: (0, 0)),
        pl.BlockSpec((1, D), lambda i: (0, 0)),
        pl.BlockSpec((D, C), lambda i: (0, 0)),
    ],
    out_specs=pl.BlockSpec((BLK, 2 * C), lambda i: (i, 0)),
    out_shape=jax.ShapeDtypeStruct((N, 2 * C), jnp.float32),
)


def _out_body(z_ref, sa_ref, sb_ref, b2_ref, o_ref):
    o = (z_ref[...] + sa_ref[...] + sb_ref[...])[:, :C] + b2_ref[...]
    m = jnp.max(o, axis=1, keepdims=True)
    e = o - m
    o_ref[...] = e - jnp.log(jnp.sum(jnp.exp(e), axis=1, keepdims=True))


_final = pl.pallas_call(
    _out_body,
    grid=(N // BLK,),
    in_specs=[
        pl.BlockSpec((BLK, 2 * C), lambda i: (i, 0)),
        pl.BlockSpec((BLK, 2 * C), lambda i: (i, 0)),
        pl.BlockSpec((BLK, 2 * C), lambda i: (i, 0)),
        pl.BlockSpec((1, C), lambda i: (0, 0)),
    ],
    out_specs=pl.BlockSpec((BLK, C), lambda i: (i, 0)),
    out_shape=jax.ShapeDtypeStruct((N, C), jnp.float32),
)


_MESH = plsc.VectorSubcoreMesh(core_axis_name="c", subcore_axis_name="s")


def _zero_acc_slice(zbuf, acc, sid, width):
    zero16 = jnp.zeros((16,), jnp.float32)

    def zrow(r, carry):
        for k in range(width // 16):
            zbuf[r, pl.ds(k * 16, 16)] = zero16
        return carry

    lax.fori_loop(0, ZROWS, zrow, 0)
    for m in range(NPT // ZROWS):
        pltpu.sync_copy(zbuf, acc.at[pl.ds(sid * NPT + m * ZROWS, ZROWS)])


def _staged_edge_loop(x_hbm, src_hbm, dst_hbm, row0, nstages, nchunks,
                      src_v, dst_v, rows_a, rows_b, acc, sem_a, sem_b):
    npairs = nchunks // 2

    for stage in range(nstages):
        base = row0 + stage * nchunks
        pltpu.sync_copy(src_hbm.at[pl.ds(base, nchunks)],
                        src_v.at[pl.ds(0, nchunks)])
        pltpu.sync_copy(dst_hbm.at[pl.ds(base, nchunks)],
                        dst_v.at[pl.ds(0, nchunks)])
        pltpu.async_copy(x_hbm.at[src_v.at[0]], rows_a, sem_a)

        def body(i, carry):
            j0 = 2 * i
            pltpu.async_copy(x_hbm.at[src_v.at[j0 + 1]], rows_b, sem_b)
            pltpu.make_async_copy(x_hbm.at[src_v.at[j0]], rows_a, sem_a).wait()
            pltpu.sync_copy(rows_a, acc.at[dst_v.at[j0]], add=True)

            @pl.when(i + 1 < npairs)
            def _():
                pltpu.async_copy(x_hbm.at[src_v.at[j0 + 2]], rows_a, sem_a)

            pltpu.make_async_copy(
                x_hbm.at[src_v.at[j0 + 1]], rows_b, sem_b).wait()
            pltpu.sync_copy(rows_b, acc.at[dst_v.at[j0 + 1]], add=True)
            return carry

        lax.fori_loop(0, npairs, body, 0)


def _seg1_body(xlo_hbm, xhi_hbm, src_hbm, dst_hbm, out_lo, out_hi,
               src_v, dst_v, rows_a, rows_b, acc, sem_a, sem_b):
    c = lax.axis_index("c")
    sid = lax.axis_index("s")

    _zero_acc_slice(rows_a, acc, sid, HALF)
    plsc.subcore_barrier()

    row0 = sid * ROWS_L1
    pl.when(c == 0)(lambda: _staged_edge_loop(
        xlo_hbm, src_hbm, dst_hbm, row0, 2, ROWS_L1 // 2,
        src_v, dst_v, rows_a, rows_b, acc, sem_a, sem_b))
    pl.when(c == 1)(lambda: _staged_edge_loop(
        xhi_hbm, src_hbm, dst_hbm, row0, 2, ROWS_L1 // 2,
        src_v, dst_v, rows_a, rows_b, acc, sem_a, sem_b))
    plsc.subcore_barrier()

    nbase = sid * NPT
    pl.when(c == 0)(lambda: pltpu.sync_copy(
        acc.at[pl.ds(nbase, NPT)], out_lo.at[pl.ds(nbase, NPT)]))
    pl.when(c == 1)(lambda: pltpu.sync_copy(
        acc.at[pl.ds(nbase, NPT)], out_hi.at[pl.ds(nbase, NPT)]))


_seg1 = pl.kernel(
    _seg1_body,
    out_type=[jax.ShapeDtypeStruct((NPAD, HALF), jnp.float32)] * 2,
    mesh=_MESH,
    scratch_types=[
        pltpu.VMEM((ROWS_L1 // 2, CHUNK), jnp.int32),
        pltpu.VMEM((ROWS_L1 // 2, CHUNK), jnp.int32),
        pltpu.VMEM((CHUNK, HALF), jnp.float32),
        pltpu.VMEM((CHUNK, HALF), jnp.float32),
        pltpu.VMEM_SHARED((NPAD, HALF), jnp.float32),
        pltpu.SemaphoreType.DMA,
        pltpu.SemaphoreType.DMA,
    ],
)


def _seg2_body(z_hbm, src_hbm, dst_hbm, out_a, out_b,
               src_v, dst_v, rows_a, rows_b, acc, sem_a, sem_b):
    c = lax.axis_index("c")
    sid = lax.axis_index("s")

    _zero_acc_slice(rows_a, acc, sid, HALF)
    plsc.subcore_barrier()

    pl.when(c == 0)(lambda: _staged_edge_loop(
        z_hbm, src_hbm, dst_hbm, sid * ROWS_L2A, 1, ROWS_L2A,
        src_v, dst_v, rows_a, rows_b, acc, sem_a, sem_b))
    pl.when(c == 1)(lambda: _staged_edge_loop(
        z_hbm, src_hbm, dst_hbm, SPLIT_L2 + sid * ROWS_L2B, 1, ROWS_L2B,
        src_v, dst_v, rows_a, rows_b, acc, sem_a, sem_b))
    plsc.subcore_barrier()

    nbase = sid * NPT
    pl.when(c == 0)(lambda: pltpu.sync_copy(
        acc.at[pl.ds(nbase, NPT)], out_a.at[pl.ds(nbase, NPT)]))
    pl.when(c == 1)(lambda: pltpu.sync_copy(
        acc.at[pl.ds(nbase, NPT)], out_b.at[pl.ds(nbase, NPT)]))


_seg2 = pl.kernel(
    _seg2_body,
    out_type=[jax.ShapeDtypeStruct((NPAD, HALF), jnp.float32)] * 2,
    mesh=_MESH,
    scratch_types=[
        pltpu.VMEM((ROWS_L2B, CHUNK), jnp.int32),
        pltpu.VMEM((ROWS_L2B, CHUNK), jnp.int32),
        pltpu.VMEM((CHUNK, HALF), jnp.float32),
        pltpu.VMEM((CHUNK, HALF), jnp.float32),
        pltpu.VMEM_SHARED((NPAD, HALF), jnp.float32),
        pltpu.SemaphoreType.DMA,
        pltpu.SemaphoreType.DMA,
    ],
)


def kernel(x, edge_index, W1, b1, W2, b2):
    e = edge_index.shape[1]
    pad = E_PAD - e
    src = jnp.concatenate(
        [edge_index[0], jnp.zeros((pad,), jnp.int32)]).reshape(ROWS, CHUNK)
    pad_dst = N + jnp.arange(pad, dtype=jnp.int32) % (NPAD - N)
    dst = jnp.concatenate([edge_index[1], pad_dst]).reshape(ROWS, CHUNK)

    xc_lo, xc_hi = _clip(x)
    s1_lo, s1_hi = _seg1(xc_lo, xc_hi, src, dst)
    z = _mid(xc_lo, xc_hi, s1_lo, s1_hi, W1, b1.reshape(1, D), W2)
    s2a, s2b = _seg2(z, src, dst)
    return _final(z, s2a, s2b, b2.reshape(1, C))


# speedup vs baseline: 1.0348x; 1.0348x over previous
import functools

import jax
import jax.numpy as jnp
from jax import lax
from jax.experimental import pallas as pl
from jax.experimental.pallas import tpu as pltpu
from jax.experimental.pallas import tpu_sc as plsc

N = 10000
D = 256
C = 64
HALF = 128
CHUNK = 128
E_PAD = 163840
ROWS = E_PAD // CHUNK
N_TILES = 16
ROWS_L1 = ROWS // N_TILES
ROWS_L2 = ROWS // 2 // N_TILES
ROWS_L2A = 64
ROWS_L2B = 16
SPLIT_L2 = ROWS_L2A * N_TILES
ROWS_L2MAX = max(ROWS_L2A, ROWS_L2B)
NPAD = 10240
NPT = NPAD // N_TILES
ZROWS = 128
BLK = 1000


def _clip_body(x_ref, lo_ref, hi_ref):
    xb = x_ref[...]
    n2 = jnp.sum(xb * xb, axis=1, keepdims=True)
    xc = xb * (1.0 / jnp.maximum(jnp.sqrt(n2), 1.0))
    lo_ref[...] = xc[:, :HALF]
    hi_ref[...] = xc[:, HALF:]


_clip = pl.pallas_call(
    _clip_body,
    grid=(N // BLK,),
    in_specs=[pl.BlockSpec((BLK, D), lambda i: (i, 0))],
    out_specs=[pl.BlockSpec((BLK, HALF), lambda i: (i, 0))] * 2,
    out_shape=[jax.ShapeDtypeStruct((N, HALF), jnp.float32)] * 2,
)


def _mid_body(lo_ref, hi_ref, slo_ref, shi_ref, w1_ref, b1_ref, w2_ref, z_ref):
    alo = lo_ref[...] + slo_ref[...]
    ahi = hi_ref[...] + shi_ref[...]
    w1 = w1_ref[...]
    h = jnp.dot(alo, w1[:HALF, :], preferred_element_type=jnp.float32)
    h = h + jnp.dot(ahi, w1[HALF:, :], preferred_element_type=jnp.float32)
    h = jnp.maximum(h + b1_ref[...], 0.0)
    n2 = jnp.sum(h * h, axis=1, keepdims=True)
    hc = h * (1.0 / jnp.maximum(jnp.sqrt(n2), 1.0))
    z = jnp.dot(hc, w2_ref[...], preferred_element_type=jnp.float32)
    z_ref[...] = jnp.concatenate([z, jnp.zeros_like(z)], axis=1)


_mid = pl.pallas_call(
    _mid_body,
    grid=(N // BLK,),
    in_specs=[
        pl.BlockSpec((BLK, HALF), lambda i: (i, 0)),
        pl.BlockSpec((BLK, HALF), lambda i: (i, 0)),
        pl.BlockSpec((BLK, HALF), lambda i: (i, 0)),
        pl.BlockSpec((BLK, HALF), lambda i: (i, 0)),
        pl.BlockSpec((D, D), lambda i: (0, 0)),
        pl.BlockSpec((1, D), lambda i: (0, 0)),
        pl.BlockSpec((D, C), lambda i: (0, 0)),
    ],
    out_specs=pl.BlockSpec((BLK, 2 * C), lambda i: (i, 0)),
    out_shape=jax.ShapeDtypeStruct((N, 2 * C), jnp.float32),
)


def _out_body(z_ref, sa_ref, sb_ref, b2_ref, o_ref):
    o = (z_ref[...] + sa_ref[...] + sb_ref[...])[:, :C] + b2_ref[...]
    m = jnp.max(o, axis=1, keepdims=True)
    e = o - m
    o_ref[...] = e - jnp.log(jnp.sum(jnp.exp(e), axis=1, keepdims=True))


_final = pl.pallas_call(
    _out_body,
    grid=(N // BLK,),
    in_specs=[
        pl.BlockSpec((BLK, 2 * C), lambda i: (i, 0)),
        pl.BlockSpec((BLK, 2 * C), lambda i: (i, 0)),
        pl.BlockSpec((BLK, 2 * C), lambda i: (i, 0)),
        pl.BlockSpec((1, C), lambda i: (0, 0)),
    ],
    out_specs=pl.BlockSpec((BLK, C), lambda i: (i, 0)),
    out_shape=jax.ShapeDtypeStruct((N, C), jnp.float32),
)


_MESH = plsc.VectorSubcoreMesh(core_axis_name="c", subcore_axis_name="s")


def _zero_acc_slice(zbuf, acc, sid, width):
    zero16 = jnp.zeros((16,), jnp.float32)

    def zrow(r, carry):
        for k in range(width // 16):
            zbuf[r, pl.ds(k * 16, 16)] = zero16
        return carry

    lax.fori_loop(0, ZROWS, zrow, 0)
    for m in range(NPT // ZROWS):
        pltpu.sync_copy(zbuf, acc.at[pl.ds(sid * NPT + m * ZROWS, ZROWS)])


def _staged_edge_loop(x_hbm, src_hbm, dst_hbm, row0, nstages, nchunks,
                      src_v, dst_v, rows_a, rows_b, acc, sem_a, sem_b):
    npairs = nchunks // 2

    for stage in range(nstages):
        base = row0 + stage * nchunks
        pltpu.sync_copy(src_hbm.at[pl.ds(base, nchunks)],
                        src_v.at[pl.ds(0, nchunks)])
        pltpu.sync_copy(dst_hbm.at[pl.ds(base, nchunks)],
                        dst_v.at[pl.ds(0, nchunks)])
        pltpu.async_copy(x_hbm.at[src_v.at[0]], rows_a, sem_a)

        def body(i, carry):
            j0 = 2 * i
            pltpu.async_copy(x_hbm.at[src_v.at[j0 + 1]], rows_b, sem_b)
            pltpu.make_async_copy(x_hbm.at[src_v.at[j0]], rows_a, sem_a).wait()
            pltpu.sync_copy(rows_a, acc.at[dst_v.at[j0]], add=True)

            @pl.when(i + 1 < npairs)
            def _():
                pltpu.async_copy(x_hbm.at[src_v.at[j0 + 2]], rows_a, sem_a)

            pltpu.make_async_copy(
                x_hbm.at[src_v.at[j0 + 1]], rows_b, sem_b).wait()
            pltpu.sync_copy(rows_b, acc.at[dst_v.at[j0 + 1]], add=True)
            return carry

        lax.fori_loop(0, npairs, body, 0)


def _seg1_body(xlo_hbm, xhi_hbm, src_hbm, dst_hbm, out_lo, out_hi,
               src_v, dst_v, rows_a, rows_b, acc, sem_a, sem_b):
    c = lax.axis_index("c")
    sid = lax.axis_index("s")

    _zero_acc_slice(rows_a, acc, sid, HALF)
    plsc.subcore_barrier()

    row0 = sid * ROWS_L1
    pl.when(c == 0)(lambda: _staged_edge_loop(
        xlo_hbm, src_hbm, dst_hbm, row0, 2, ROWS_L1 // 2,
        src_v, dst_v, rows_a, rows_b, acc, sem_a, sem_b))
    pl.when(c == 1)(lambda: _staged_edge_loop(
        xhi_hbm, src_hbm, dst_hbm, row0, 2, ROWS_L1 // 2,
        src_v, dst_v, rows_a, rows_b, acc, sem_a, sem_b))
    plsc.subcore_barrier()

    nbase = sid * NPT
    pl.when(c == 0)(lambda: pltpu.sync_copy(
        acc.at[pl.ds(nbase, NPT)], out_lo.at[pl.ds(nbase, NPT)]))
    pl.when(c == 1)(lambda: pltpu.sync_copy(
        acc.at[pl.ds(nbase, NPT)], out_hi.at[pl.ds(nbase, NPT)]))


_seg1 = pl.kernel(
    _seg1_body,
    out_type=[jax.ShapeDtypeStruct((NPAD, HALF), jnp.float32)] * 2,
    mesh=_MESH,
    scratch_types=[
        pltpu.VMEM((ROWS_L1 // 2, CHUNK), jnp.int32),
        pltpu.VMEM((ROWS_L1 // 2, CHUNK), jnp.int32),
        pltpu.VMEM((CHUNK, HALF), jnp.float32),
        pltpu.VMEM((CHUNK, HALF), jnp.float32),
        pltpu.VMEM_SHARED((NPAD, HALF), jnp.float32),
        pltpu.SemaphoreType.DMA,
        pltpu.SemaphoreType.DMA,
    ],
)


def _seg2_body(z_hbm, src_hbm, dst_hbm, out_a, out_b,
               src_v, dst_v, rows_a, rows_b, acc, sem_a, sem_b):
    c = lax.axis_index("c")
    sid = lax.axis_index("s")

    _zero_acc_slice(rows_a, acc, sid, HALF)
    plsc.subcore_barrier()

    pl.when(c == 0)(lambda: _staged_edge_loop(
        z_hbm, src_hbm, dst_hbm, sid * ROWS_L2A, 1, ROWS_L2A,
        src_v, dst_v, rows_a, rows_b, acc, sem_a, sem_b))
    pl.when(c == 1)(lambda: _staged_edge_loop(
        z_hbm, src_hbm, dst_hbm, SPLIT_L2 + sid * ROWS_L2B, 1, ROWS_L2B,
        src_v, dst_v, rows_a, rows_b, acc, sem_a, sem_b))
    plsc.subcore_barrier()

    nbase = sid * NPT
    pl.when(c == 0)(lambda: pltpu.sync_copy(
        acc.at[pl.ds(nbase, NPT)], out_a.at[pl.ds(nbase, NPT)]))
    pl.when(c == 1)(lambda: pltpu.sync_copy(
        acc.at[pl.ds(nbase, NPT)], out_b.at[pl.ds(nbase, NPT)]))


_seg2 = pl.kernel(
    _seg2_body,
    out_type=[jax.ShapeDtypeStruct((NPAD, HALF), jnp.float32)] * 2,
    mesh=_MESH,
    scratch_types=[
        pltpu.VMEM((ROWS_L2MAX, CHUNK), jnp.int32),
        pltpu.VMEM((ROWS_L2MAX, CHUNK), jnp.int32),
        pltpu.VMEM((CHUNK, HALF), jnp.float32),
        pltpu.VMEM((CHUNK, HALF), jnp.float32),
        pltpu.VMEM_SHARED((NPAD, HALF), jnp.float32),
        pltpu.SemaphoreType.DMA,
        pltpu.SemaphoreType.DMA,
    ],
)


def kernel(x, edge_index, W1, b1, W2, b2):
    e = edge_index.shape[1]
    pad = E_PAD - e
    src = jnp.concatenate(
        [edge_index[0], jnp.zeros((pad,), jnp.int32)]).reshape(ROWS, CHUNK)
    pad_dst = N + jnp.arange(pad, dtype=jnp.int32) % (NPAD - N)
    dst = jnp.concatenate([edge_index[1], pad_dst]).reshape(ROWS, CHUNK)

    xc_lo, xc_hi = _clip(x)
    s1_lo, s1_hi = _seg1(xc_lo, xc_hi, src, dst)
    z = _mid(xc_lo, xc_hi, s1_lo, s1_hi, W1, b1.reshape(1, D), W2)
    s2a, s2b = _seg2(z, src, dst)
    return _final(z, s2a, s2b, b2.reshape(1, C))


# speedup vs baseline: 2.6551x; 2.5658x over previous
import functools

import jax
import jax.numpy as jnp
from jax import lax
from jax.experimental import pallas as pl
from jax.experimental.pallas import tpu as pltpu
from jax.experimental.pallas import tpu_sc as plsc

N = 10000
D = 256
C = 64
HALF = 128
CHUNK = 128
E_PAD = 163840
ROWS = E_PAD // CHUNK
N_TILES = 16
ROWS_L1 = ROWS // N_TILES
ROWS_L2 = ROWS // 2 // N_TILES
ROWS_L2A = 40
ROWS_L2B = 40
SPLIT_L2 = ROWS_L2A * N_TILES
ROWS_L2MAX = max(ROWS_L2A, ROWS_L2B)
NPAD = 10240
NPT = NPAD // N_TILES
ZROWS = 128
BLK = 1000


def _clip_body(x_ref, lo_ref, hi_ref):
    xb = x_ref[...]
    n2 = jnp.sum(xb * xb, axis=1, keepdims=True)
    xc = xb * (1.0 / jnp.maximum(jnp.sqrt(n2), 1.0))
    lo_ref[...] = xc[:, :HALF]
    hi_ref[...] = xc[:, HALF:]


_clip = pl.pallas_call(
    _clip_body,
    grid=(N // BLK,),
    in_specs=[pl.BlockSpec((BLK, D), lambda i: (i, 0))],
    out_specs=[pl.BlockSpec((BLK, HALF), lambda i: (i, 0))] * 2,
    out_shape=[jax.ShapeDtypeStruct((N, HALF), jnp.float32)] * 2,
)


def _mid_body(lo_ref, hi_ref, slo_ref, shi_ref, w1_ref, b1_ref, w2_ref, z_ref):
    alo = lo_ref[...] + slo_ref[...]
    ahi = hi_ref[...] + shi_ref[...]
    w1 = w1_ref[...]
    h = jnp.dot(alo, w1[:HALF, :], preferred_element_type=jnp.float32)
    h = h + jnp.dot(ahi, w1[HALF:, :], preferred_element_type=jnp.float32)
    h = jnp.maximum(h + b1_ref[...], 0.0)
    n2 = jnp.sum(h * h, axis=1, keepdims=True)
    hc = h * (1.0 / jnp.maximum(jnp.sqrt(n2), 1.0))
    z = jnp.dot(hc, w2_ref[...], preferred_element_type=jnp.float32)
    z_ref[...] = jnp.concatenate([z, jnp.zeros_like(z)], axis=1)


_mid = pl.pallas_call(
    _mid_body,
    grid=(N // BLK,),
    in_specs=[
        pl.BlockSpec((BLK, HALF), lambda i: (i, 0)),
        pl.BlockSpec((BLK, HALF), lambda i: (i, 0)),
        pl.BlockSpec((BLK, HALF), lambda i: (i, 0)),
        pl.BlockSpec((BLK, HALF), lambda i: (i, 0)),
        pl.BlockSpec((D, D), lambda i: (0, 0)),
        pl.BlockSpec((1, D), lambda i: (0, 0)),
        pl.BlockSpec((D, C), lambda i: (0, 0)),
    ],
    out_specs=pl.BlockSpec((BLK, 2 * C), lambda i: (i, 0)),
    out_shape=jax.ShapeDtypeStruct((N, 2 * C), jnp.float32),
)


def _out_body(z_ref, sa_ref, sb_ref, b2_ref, o_ref):
    o = (z_ref[...] + sa_ref[...] + sb_ref[...])[:, :C] + b2_ref[...]
    m = jnp.max(o, axis=1, keepdims=True)
    e = o - m
    o_ref[...] = e - jnp.log(jnp.sum(jnp.exp(e), axis=1, keepdims=True))


_final = pl.pallas_call(
    _out_body,
    grid=(N // BLK,),
    in_specs=[
        pl.BlockSpec((BLK, 2 * C), lambda i: (i, 0)),
        pl.BlockSpec((BLK, 2 * C), lambda i: (i, 0)),
        pl.BlockSpec((BLK, 2 * C), lambda i: (i, 0)),
        pl.BlockSpec((1, C), lambda i: (0, 0)),
    ],
    out_specs=pl.BlockSpec((BLK, C), lambda i: (i, 0)),
    out_shape=jax.ShapeDtypeStruct((N, C), jnp.float32),
)


_MESH = plsc.VectorSubcoreMesh(core_axis_name="c", subcore_axis_name="s")


def _zero_acc_slice(zbuf, acc, sid, width):
    zero16 = jnp.zeros((16,), jnp.float32)

    def zrow(r, carry):
        for k in range(width // 16):
            zbuf[r, pl.ds(k * 16, 16)] = zero16
        return carry

    lax.fori_loop(0, ZROWS, zrow, 0)
    for m in range(NPT // ZROWS):
        pltpu.sync_copy(zbuf, acc.at[pl.ds(sid * NPT + m * ZROWS, ZROWS)])


def _staged_edge_loop(x_hbm, src_hbm, dst_hbm, row0, nstages, nchunks,
                      src_v, dst_v, rows_a, rows_b, acc, sem_a, sem_b):
    npairs = nchunks // 2

    for stage in range(nstages):
        base = row0 + stage * nchunks
        pltpu.sync_copy(src_hbm.at[pl.ds(base, nchunks)],
                        src_v.at[pl.ds(0, nchunks)])
        pltpu.sync_copy(dst_hbm.at[pl.ds(base, nchunks)],
                        dst_v.at[pl.ds(0, nchunks)])
        pltpu.async_copy(x_hbm.at[src_v.at[0]], rows_a, sem_a)

        def body(i, carry):
            j0 = 2 * i
            pltpu.async_copy(x_hbm.at[src_v.at[j0 + 1]], rows_b, sem_b)
            pltpu.make_async_copy(x_hbm.at[src_v.at[j0]], rows_a, sem_a).wait()
            pltpu.sync_copy(rows_a, acc.at[dst_v.at[j0]], add=True)

            @pl.when(i + 1 < npairs)
            def _():
                pltpu.async_copy(x_hbm.at[src_v.at[j0 + 2]], rows_a, sem_a)

            pltpu.make_async_copy(
                x_hbm.at[src_v.at[j0 + 1]], rows_b, sem_b).wait()
            pltpu.sync_copy(rows_b, acc.at[dst_v.at[j0 + 1]], add=True)
            return carry

        lax.fori_loop(0, npairs, body, 0)


def _seg1_body(xlo_hbm, xhi_hbm, src_hbm, dst_hbm, out_lo, out_hi,
               src_v, dst_v, rows_a, rows_b, acc, sem_a, sem_b):
    c = lax.axis_index("c")
    sid = lax.axis_index("s")

    _zero_acc_slice(rows_a, acc, sid, HALF)
    plsc.subcore_barrier()

    row0 = sid * ROWS_L1
    pl.when(c == 0)(lambda: _staged_edge_loop(
        xlo_hbm, src_hbm, dst_hbm, row0, 2, ROWS_L1 // 2,
        src_v, dst_v, rows_a, rows_b, acc, sem_a, sem_b))
    pl.when(c == 1)(lambda: _staged_edge_loop(
        xhi_hbm, src_hbm, dst_hbm, row0, 2, ROWS_L1 // 2,
        src_v, dst_v, rows_a, rows_b, acc, sem_a, sem_b))
    plsc.subcore_barrier()

    nbase = sid * NPT
    pl.when(c == 0)(lambda: pltpu.sync_copy(
        acc.at[pl.ds(nbase, NPT)], out_lo.at[pl.ds(nbase, NPT)]))
    pl.when(c == 1)(lambda: pltpu.sync_copy(
        acc.at[pl.ds(nbase, NPT)], out_hi.at[pl.ds(nbase, NPT)]))


_seg1 = pl.kernel(
    _seg1_body,
    out_type=[jax.ShapeDtypeStruct((NPAD, HALF), jnp.float32)] * 2,
    mesh=_MESH,
    scratch_types=[
        pltpu.VMEM((ROWS_L1 // 2, CHUNK), jnp.int32),
        pltpu.VMEM((ROWS_L1 // 2, CHUNK), jnp.int32),
        pltpu.VMEM((CHUNK, HALF), jnp.float32),
        pltpu.VMEM((CHUNK, HALF), jnp.float32),
        pltpu.VMEM_SHARED((NPAD, HALF), jnp.float32),
        pltpu.SemaphoreType.DMA,
        pltpu.SemaphoreType.DMA,
    ],
)


def _seg2_body(z_hbm, src_hbm, dst_hbm, out_a, out_b,
               src_v, dst_v, rows_a, rows_b, acc, sem_a, sem_b):
    c = lax.axis_index("c")
    sid = lax.axis_index("s")

    _zero_acc_slice(rows_a, acc, sid, HALF)
    plsc.subcore_barrier()

    pl.when(c == 0)(lambda: _staged_edge_loop(
        z_hbm, src_hbm, dst_hbm, sid * ROWS_L2A, 1, ROWS_L2A,
        src_v, dst_v, rows_a, rows_b, acc, sem_a, sem_b))
    pl.when(c == 1)(lambda: _staged_edge_loop(
        z_hbm, src_hbm, dst_hbm, SPLIT_L2 + sid * ROWS_L2B, 1, ROWS_L2B,
        src_v, dst_v, rows_a, rows_b, acc, sem_a, sem_b))
    plsc.subcore_barrier()

    nbase = sid * NPT
    pl.when(c == 0)(lambda: pltpu.sync_copy(
        acc.at[pl.ds(nbase, NPT)], out_a.at[pl.ds(nbase, NPT)]))
    pl.when(c == 1)(lambda: pltpu.sync_copy(
        acc.at[pl.ds(nbase, NPT)], out_b.at[pl.ds(nbase, NPT)]))


_seg2 = pl.kernel(
    _seg2_body,
    out_type=[jax.ShapeDtypeStruct((NPAD, HALF), jnp.float32)] * 2,
    mesh=_MESH,
    scratch_types=[
        pltpu.VMEM((ROWS_L2MAX, CHUNK), jnp.int32),
        pltpu.VMEM((ROWS_L2MAX, CHUNK), jnp.int32),
        pltpu.VMEM((CHUNK, HALF), jnp.float32),
        pltpu.VMEM((CHUNK, HALF), jnp.float32),
        pltpu.VMEM_SHARED((NPAD, HALF), jnp.float32),
        pltpu.SemaphoreType.DMA,
        pltpu.SemaphoreType.DMA,
    ],
)


def kernel(x, edge_index, W1, b1, W2, b2):
    e = edge_index.shape[1]
    pad = E_PAD - e
    pad_src = jnp.arange(pad, dtype=jnp.int32) % N
    src = jnp.concatenate([edge_index[0], pad_src]).reshape(ROWS, CHUNK)
    pad_dst = N + jnp.arange(pad, dtype=jnp.int32) % (NPAD - N)
    dst = jnp.concatenate([edge_index[1], pad_dst]).reshape(ROWS, CHUNK)

    xc_lo, xc_hi = _clip(x)
    s1_lo, s1_hi = _seg1(xc_lo, xc_hi, src, dst)
    z = _mid(xc_lo, xc_hi, s1_lo, s1_hi, W1, b1.reshape(1, D), W2)
    s2a, s2b = _seg2(z, src, dst)
    return _final(z, s2a, s2b, b2.reshape(1, C))


# speedup vs baseline: 2.7419x; 1.0327x over previous
import functools

import numpy as np
import jax
import jax.numpy as jnp
from jax import lax
from jax.experimental import pallas as pl
from jax.experimental.pallas import tpu as pltpu
from jax.experimental.pallas import tpu_sc as plsc

N = 10000
D = 256
C = 64
HALF = 128
CHUNK = 128
E_PAD = 163840
ROWS = E_PAD // CHUNK
N_TILES = 16
ROWS_L1 = ROWS // N_TILES
ROWS_L2 = ROWS // 2 // N_TILES
ROWS_L2A = 40
ROWS_L2B = 40
SPLIT_L2 = ROWS_L2A * N_TILES
ROWS_L2MAX = max(ROWS_L2A, ROWS_L2B)
NPAD = 10240
NPT = NPAD // N_TILES
ZROWS = 128
BLK = 2000


def _clip_body(x_ref, lo_ref, hi_ref):
    xb = x_ref[...]
    n2 = jnp.sum(xb * xb, axis=1, keepdims=True)
    xc = xb * (1.0 / jnp.maximum(jnp.sqrt(n2), 1.0))
    lo_ref[...] = xc[:, :HALF]
    hi_ref[...] = xc[:, HALF:]


_clip = pl.pallas_call(
    _clip_body,
    grid=(N // BLK,),
    in_specs=[pl.BlockSpec((BLK, D), lambda i: (i, 0))],
    out_specs=[pl.BlockSpec((BLK, HALF), lambda i: (i, 0))] * 2,
    out_shape=[jax.ShapeDtypeStruct((N, HALF), jnp.float32)] * 2,
)


def _mid_body(lo_ref, hi_ref, slo_ref, shi_ref, w1_ref, b1_ref, w2_ref, z_ref):
    alo = lo_ref[...] + slo_ref[...]
    ahi = hi_ref[...] + shi_ref[...]
    w1 = w1_ref[...]
    h = jnp.dot(alo, w1[:HALF, :], preferred_element_type=jnp.float32)
    h = h + jnp.dot(ahi, w1[HALF:, :], preferred_element_type=jnp.float32)
    h = jnp.maximum(h + b1_ref[...], 0.0)
    n2 = jnp.sum(h * h, axis=1, keepdims=True)
    hc = h * (1.0 / jnp.maximum(jnp.sqrt(n2), 1.0))
    z = jnp.dot(hc, w2_ref[...], preferred_element_type=jnp.float32)
    z_ref[...] = jnp.concatenate([z, jnp.zeros_like(z)], axis=1)


_mid = pl.pallas_call(
    _mid_body,
    grid=(N // BLK,),
    in_specs=[
        pl.BlockSpec((BLK, HALF), lambda i: (i, 0)),
        pl.BlockSpec((BLK, HALF), lambda i: (i, 0)),
        pl.BlockSpec((BLK, HALF), lambda i: (i, 0)),
        pl.BlockSpec((BLK, HALF), lambda i: (i, 0)),
        pl.BlockSpec((D, D), lambda i: (0, 0)),
        pl.BlockSpec((1, D), lambda i: (0, 0)),
        pl.BlockSpec((D, C), lambda i: (0, 0)),
    ],
    out_specs=pl.BlockSpec((BLK, 2 * C), lambda i: (i, 0)),
    out_shape=jax.ShapeDtypeStruct((N, 2 * C), jnp.float32),
)


def _out_body(z_ref, sa_ref, sb_ref, b2_ref, o_ref):
    o = (z_ref[...] + sa_ref[...] + sb_ref[...])[:, :C] + b2_ref[...]
    m = jnp.max(o, axis=1, keepdims=True)
    e = o - m
    o_ref[...] = e - jnp.log(jnp.sum(jnp.exp(e), axis=1, keepdims=True))


_final = pl.pallas_call(
    _out_body,
    grid=(N // BLK,),
    in_specs=[
        pl.BlockSpec((BLK, 2 * C), lambda i: (i, 0)),
        pl.BlockSpec((BLK, 2 * C), lambda i: (i, 0)),
        pl.BlockSpec((BLK, 2 * C), lambda i: (i, 0)),
        pl.BlockSpec((1, C), lambda i: (0, 0)),
    ],
    out_specs=pl.BlockSpec((BLK, C), lambda i: (i, 0)),
    out_shape=jax.ShapeDtypeStruct((N, C), jnp.float32),
)


_MESH = plsc.VectorSubcoreMesh(core_axis_name="c", subcore_axis_name="s")


def _zero_acc_slice(zbuf, acc, sid, width):
    zero16 = jnp.zeros((16,), jnp.float32)

    def zrow(r, carry):
        for k in range(width // 16):
            zbuf[r, pl.ds(k * 16, 16)] = zero16
        return carry

    lax.fori_loop(0, ZROWS, zrow, 0)
    for m in range(NPT // ZROWS):
        pltpu.sync_copy(zbuf, acc.at[pl.ds(sid * NPT + m * ZROWS, ZROWS)])


def _staged_edge_loop(x_hbm, src_hbm, dst_hbm, row0, nstages, nchunks,
                      src_v, dst_v, rows_a, rows_b, acc, sem_a, sem_b):
    npairs = nchunks // 2

    for stage in range(nstages):
        base = row0 + stage * nchunks
        pltpu.sync_copy(src_hbm.at[pl.ds(base, nchunks)],
                        src_v.at[pl.ds(0, nchunks)])
        pltpu.sync_copy(dst_hbm.at[pl.ds(base, nchunks)],
                        dst_v.at[pl.ds(0, nchunks)])
        pltpu.async_copy(x_hbm.at[src_v.at[0]], rows_a, sem_a)

        def body(i, carry):
            j0 = 2 * i
            pltpu.async_copy(x_hbm.at[src_v.at[j0 + 1]], rows_b, sem_b)
            pltpu.make_async_copy(x_hbm.at[src_v.at[j0]], rows_a, sem_a).wait()
            pltpu.sync_copy(rows_a, acc.at[dst_v.at[j0]], add=True)

            @pl.when(i + 1 < npairs)
            def _():
                pltpu.async_copy(x_hbm.at[src_v.at[j0 + 2]], rows_a, sem_a)

            pltpu.make_async_copy(
                x_hbm.at[src_v.at[j0 + 1]], rows_b, sem_b).wait()
            pltpu.sync_copy(rows_b, acc.at[dst_v.at[j0 + 1]], add=True)
            return carry

        lax.fori_loop(0, npairs, body, 0)


def _seg1_body(xlo_hbm, xhi_hbm, src_hbm, dst_hbm, out_lo, out_hi,
               src_v, dst_v, rows_a, rows_b, acc, sem_a, sem_b):
    c = lax.axis_index("c")
    sid = lax.axis_index("s")

    _zero_acc_slice(rows_a, acc, sid, HALF)
    plsc.subcore_barrier()

    row0 = sid * ROWS_L1
    pl.when(c == 0)(lambda: _staged_edge_loop(
        xlo_hbm, src_hbm, dst_hbm, row0, 2, ROWS_L1 // 2,
        src_v, dst_v, rows_a, rows_b, acc, sem_a, sem_b))
    pl.when(c == 1)(lambda: _staged_edge_loop(
        xhi_hbm, src_hbm, dst_hbm, row0, 2, ROWS_L1 // 2,
        src_v, dst_v, rows_a, rows_b, acc, sem_a, sem_b))
    plsc.subcore_barrier()

    nbase = sid * NPT
    pl.when(c == 0)(lambda: pltpu.sync_copy(
        acc.at[pl.ds(nbase, NPT)], out_lo.at[pl.ds(nbase, NPT)]))
    pl.when(c == 1)(lambda: pltpu.sync_copy(
        acc.at[pl.ds(nbase, NPT)], out_hi.at[pl.ds(nbase, NPT)]))


_seg1 = pl.kernel(
    _seg1_body,
    out_type=[jax.ShapeDtypeStruct((NPAD, HALF), jnp.float32)] * 2,
    mesh=_MESH,
    scratch_types=[
        pltpu.VMEM((ROWS_L1 // 2, CHUNK), jnp.int32),
        pltpu.VMEM((ROWS_L1 // 2, CHUNK), jnp.int32),
        pltpu.VMEM((CHUNK, HALF), jnp.float32),
        pltpu.VMEM((CHUNK, HALF), jnp.float32),
        pltpu.VMEM_SHARED((NPAD, HALF), jnp.float32),
        pltpu.SemaphoreType.DMA,
        pltpu.SemaphoreType.DMA,
    ],
)


def _seg2_body(z_hbm, src_hbm, dst_hbm, out_a, out_b,
               src_v, dst_v, rows_a, rows_b, acc, sem_a, sem_b):
    c = lax.axis_index("c")
    sid = lax.axis_index("s")

    _zero_acc_slice(rows_a, acc, sid, HALF)
    plsc.subcore_barrier()

    pl.when(c == 0)(lambda: _staged_edge_loop(
        z_hbm, src_hbm, dst_hbm, sid * ROWS_L2A, 1, ROWS_L2A,
        src_v, dst_v, rows_a, rows_b, acc, sem_a, sem_b))
    pl.when(c == 1)(lambda: _staged_edge_loop(
        z_hbm, src_hbm, dst_hbm, SPLIT_L2 + sid * ROWS_L2B, 1, ROWS_L2B,
        src_v, dst_v, rows_a, rows_b, acc, sem_a, sem_b))
    plsc.subcore_barrier()

    nbase = sid * NPT
    pl.when(c == 0)(lambda: pltpu.sync_copy(
        acc.at[pl.ds(nbase, NPT)], out_a.at[pl.ds(nbase, NPT)]))
    pl.when(c == 1)(lambda: pltpu.sync_copy(
        acc.at[pl.ds(nbase, NPT)], out_b.at[pl.ds(nbase, NPT)]))


_seg2 = pl.kernel(
    _seg2_body,
    out_type=[jax.ShapeDtypeStruct((NPAD, HALF), jnp.float32)] * 2,
    mesh=_MESH,
    scratch_types=[
        pltpu.VMEM((ROWS_L2MAX, CHUNK), jnp.int32),
        pltpu.VMEM((ROWS_L2MAX, CHUNK), jnp.int32),
        pltpu.VMEM((CHUNK, HALF), jnp.float32),
        pltpu.VMEM((CHUNK, HALF), jnp.float32),
        pltpu.VMEM_SHARED((NPAD, HALF), jnp.float32),
        pltpu.SemaphoreType.DMA,
        pltpu.SemaphoreType.DMA,
    ],
)


def kernel(x, edge_index, W1, b1, W2, b2):
    e = edge_index.shape[1]
    pad = E_PAD - e
    pad_src = jnp.asarray(np.arange(pad, dtype=np.int32) % N)
    pad_dst = jnp.asarray(N + np.arange(pad, dtype=np.int32) % (NPAD - N))
    src = jnp.concatenate([edge_index[0], pad_src]).reshape(ROWS, CHUNK)
    dst = jnp.concatenate([edge_index[1], pad_dst]).reshape(ROWS, CHUNK)

    xc_lo, xc_hi = _clip(x)
    s1_lo, s1_hi = _seg1(xc_lo, xc_hi, src, dst)
    z = _mid(xc_lo, xc_hi, s1_lo, s1_hi, W1, b1.reshape(1, D), W2)
    s2a, s2b = _seg2(z, src, dst)
    return _final(z, s2a, s2b, b2.reshape(1, C))


# speedup vs baseline: 2.9221x; 1.0657x over previous
import functools

import numpy as np
import jax
import jax.numpy as jnp
from jax import lax
from jax.experimental import pallas as pl
from jax.experimental.pallas import tpu as pltpu
from jax.experimental.pallas import tpu_sc as plsc

N = 10000
D = 256
C = 64
HALF = 128
CHUNK = 128
E_PAD = 163840
ROWS = E_PAD // CHUNK
N_TILES = 16
ROWS_L1 = ROWS // N_TILES
ROWS_L2 = ROWS // 2 // N_TILES
ROWS_L2A = 40
ROWS_L2B = 40
SPLIT_L2 = ROWS_L2A * N_TILES
ROWS_L2MAX = max(ROWS_L2A, ROWS_L2B)
NPAD = 10240
NPT = NPAD // N_TILES
ZROWS = 128
BLK = 2000


def _clip_body(x_ref, lo_ref, hi_ref):
    xb = x_ref[...]
    n2 = jnp.sum(xb * xb, axis=1, keepdims=True)
    xc = xb * (1.0 / jnp.maximum(jnp.sqrt(n2), 1.0))
    lo_ref[...] = xc[:, :HALF]
    hi_ref[...] = xc[:, HALF:]


_clip = pl.pallas_call(
    _clip_body,
    grid=(N // BLK,),
    in_specs=[pl.BlockSpec((BLK, D), lambda i: (i, 0))],
    out_specs=[pl.BlockSpec((BLK, HALF), lambda i: (i, 0))] * 2,
    out_shape=[jax.ShapeDtypeStruct((N, HALF), jnp.float32)] * 2,
)


def _mid_body(lo_ref, hi_ref, slo_ref, shi_ref, w1_ref, b1_ref, w2_ref, z_ref):
    alo = lo_ref[...] + slo_ref[...]
    ahi = hi_ref[...] + shi_ref[...]
    w1 = w1_ref[...]
    h = jnp.dot(alo, w1[:HALF, :], preferred_element_type=jnp.float32)
    h = h + jnp.dot(ahi, w1[HALF:, :], preferred_element_type=jnp.float32)
    h = jnp.maximum(h + b1_ref[...], 0.0)
    n2 = jnp.sum(h * h, axis=1, keepdims=True)
    hc = h * (1.0 / jnp.maximum(jnp.sqrt(n2), 1.0))
    z_ref[...] = jnp.dot(hc, w2_ref[...], preferred_element_type=jnp.float32)


_mid = pl.pallas_call(
    _mid_body,
    grid=(N // BLK,),
    in_specs=[
        pl.BlockSpec((BLK, HALF), lambda i: (i, 0)),
        pl.BlockSpec((BLK, HALF), lambda i: (i, 0)),
        pl.BlockSpec((BLK, HALF), lambda i: (i, 0)),
        pl.BlockSpec((BLK, HALF), lambda i: (i, 0)),
        pl.BlockSpec((D, D), lambda i: (0, 0)),
        pl.BlockSpec((1, D), lambda i: (0, 0)),
        pl.BlockSpec((D, C), lambda i: (0, 0)),
    ],
    out_specs=pl.BlockSpec((BLK, C), lambda i: (i, 0)),
    out_shape=jax.ShapeDtypeStruct((NPAD, C), jnp.float32),
)


def _out_body(z_ref, sa_ref, sb_ref, b2_ref, o_ref):
    o = z_ref[...] + sa_ref[...] + sb_ref[...] + b2_ref[...]
    parts = []
    for h in range(2):
        oh = o[:, h * C:(h + 1) * C]
        m = jnp.max(oh, axis=1, keepdims=True)
        e = oh - m
        parts.append(e - jnp.log(jnp.sum(jnp.exp(e), axis=1, keepdims=True)))
    o_ref[...] = jnp.concatenate(parts, axis=1)


BLK2 = 1024

_final = pl.pallas_call(
    _out_body,
    grid=(NPAD // 2 // BLK2,),
    in_specs=[
        pl.BlockSpec((BLK2, 2 * C), lambda i: (i, 0)),
        pl.BlockSpec((BLK2, 2 * C), lambda i: (i, 0)),
        pl.BlockSpec((BLK2, 2 * C), lambda i: (i, 0)),
        pl.BlockSpec((1, 2 * C), lambda i: (0, 0)),
    ],
    out_specs=pl.BlockSpec((BLK2, 2 * C), lambda i: (i, 0)),
    out_shape=jax.ShapeDtypeStruct((NPAD // 2, 2 * C), jnp.float32),
)


_MESH = plsc.VectorSubcoreMesh(core_axis_name="c", subcore_axis_name="s")


def _zero_acc_slice(zbuf, acc, sid, width):
    zero16 = jnp.zeros((16,), jnp.float32)

    def zrow(r, carry):
        for k in range(width // 16):
            zbuf[r, pl.ds(k * 16, 16)] = zero16
        return carry

    lax.fori_loop(0, ZROWS, zrow, 0)
    for m in range(NPT // ZROWS):
        pltpu.sync_copy(zbuf, acc.at[pl.ds(sid * NPT + m * ZROWS, ZROWS)])


def _staged_edge_loop(x_hbm, src_hbm, dst_hbm, row0, nstages, nchunks,
                      src_v, dst_v, rows_a, rows_b, acc, sem_a, sem_b):
    npairs = nchunks // 2

    for stage in range(nstages):
        base = row0 + stage * nchunks
        pltpu.sync_copy(src_hbm.at[pl.ds(base, nchunks)],
                        src_v.at[pl.ds(0, nchunks)])
        pltpu.sync_copy(dst_hbm.at[pl.ds(base, nchunks)],
                        dst_v.at[pl.ds(0, nchunks)])
        pltpu.async_copy(x_hbm.at[src_v.at[0]], rows_a, sem_a)

        def body(i, carry):
            j0 = 2 * i
            pltpu.async_copy(x_hbm.at[src_v.at[j0 + 1]], rows_b, sem_b)
            pltpu.make_async_copy(x_hbm.at[src_v.at[j0]], rows_a, sem_a).wait()
            pltpu.sync_copy(rows_a, acc.at[dst_v.at[j0]], add=True)

            @pl.when(i + 1 < npairs)
            def _():
                pltpu.async_copy(x_hbm.at[src_v.at[j0 + 2]], rows_a, sem_a)

            pltpu.make_async_copy(
                x_hbm.at[src_v.at[j0 + 1]], rows_b, sem_b).wait()
            pltpu.sync_copy(rows_b, acc.at[dst_v.at[j0 + 1]], add=True)
            return carry

        lax.fori_loop(0, npairs, body, 0)


def _seg1_body(xlo_hbm, xhi_hbm, src_hbm, dst_hbm, out_lo, out_hi,
               src_v, dst_v, rows_a, rows_b, acc, sem_a, sem_b):
    c = lax.axis_index("c")
    sid = lax.axis_index("s")

    _zero_acc_slice(rows_a, acc, sid, HALF)
    plsc.subcore_barrier()

    row0 = sid * ROWS_L1
    pl.when(c == 0)(lambda: _staged_edge_loop(
        xlo_hbm, src_hbm, dst_hbm, row0, 2, ROWS_L1 // 2,
        src_v, dst_v, rows_a, rows_b, acc, sem_a, sem_b))
    pl.when(c == 1)(lambda: _staged_edge_loop(
        xhi_hbm, src_hbm, dst_hbm, row0, 2, ROWS_L1 // 2,
        src_v, dst_v, rows_a, rows_b, acc, sem_a, sem_b))
    plsc.subcore_barrier()

    nbase = sid * NPT
    pl.when(c == 0)(lambda: pltpu.sync_copy(
        acc.at[pl.ds(nbase, NPT)], out_lo.at[pl.ds(nbase, NPT)]))
    pl.when(c == 1)(lambda: pltpu.sync_copy(
        acc.at[pl.ds(nbase, NPT)], out_hi.at[pl.ds(nbase, NPT)]))


_seg1 = pl.kernel(
    _seg1_body,
    out_type=[jax.ShapeDtypeStruct((NPAD, HALF), jnp.float32)] * 2,
    mesh=_MESH,
    scratch_types=[
        pltpu.VMEM((ROWS_L1 // 2, CHUNK), jnp.int32),
        pltpu.VMEM((ROWS_L1 // 2, CHUNK), jnp.int32),
        pltpu.VMEM((CHUNK, HALF), jnp.float32),
        pltpu.VMEM((CHUNK, HALF), jnp.float32),
        pltpu.VMEM_SHARED((NPAD, HALF), jnp.float32),
        pltpu.SemaphoreType.DMA,
        pltpu.SemaphoreType.DMA,
    ],
)


def _seg2_body(z_hbm, src_hbm, dst_hbm, out_a, out_b,
               src_v, dst_v, rows_a, rows_b, acc, sem_a, sem_b):
    c = lax.axis_index("c")
    sid = lax.axis_index("s")
    z2d = z_hbm

    _zero_acc_slice(rows_a, acc, sid, C)
    plsc.subcore_barrier()

    pl.when(c == 0)(lambda: _staged_edge_loop(
        z2d, src_hbm, dst_hbm, sid * ROWS_L2A, 1, ROWS_L2A,
        src_v, dst_v, rows_a, rows_b, acc, sem_a, sem_b))
    pl.when(c == 1)(lambda: _staged_edge_loop(
        z2d, src_hbm, dst_hbm, SPLIT_L2 + sid * ROWS_L2B, 1, ROWS_L2B,
        src_v, dst_v, rows_a, rows_b, acc, sem_a, sem_b))
    plsc.subcore_barrier()

    nbase = sid * NPT
    pl.when(c == 0)(lambda: pltpu.sync_copy(
        acc.at[pl.ds(nbase, NPT)], out_a.at[pl.ds(nbase, NPT)]))
    pl.when(c == 1)(lambda: pltpu.sync_copy(
        acc.at[pl.ds(nbase, NPT)], out_b.at[pl.ds(nbase, NPT)]))


_seg2 = pl.kernel(
    _seg2_body,
    out_type=[jax.ShapeDtypeStruct((NPAD, C), jnp.float32)] * 2,
    mesh=_MESH,
    scratch_types=[
        pltpu.VMEM((ROWS_L2MAX, CHUNK), jnp.int32),
        pltpu.VMEM((ROWS_L2MAX, CHUNK), jnp.int32),
        pltpu.VMEM((CHUNK, C), jnp.float32),
        pltpu.VMEM((CHUNK, C), jnp.float32),
        pltpu.VMEM_SHARED((NPAD, C), jnp.float32),
        pltpu.SemaphoreType.DMA,
        pltpu.SemaphoreType.DMA,
    ],
    compiler_params=pltpu.CompilerParams(use_tc_tiling_on_sc=False),
)


def kernel(x, edge_index, W1, b1, W2, b2):
    e = edge_index.shape[1]
    pad = E_PAD - e
    pad_src = jnp.asarray(np.arange(pad, dtype=np.int32) % N)
    pad_dst = jnp.asarray(N + np.arange(pad, dtype=np.int32) % (NPAD - N))
    src = jnp.concatenate([edge_index[0], pad_src]).reshape(ROWS, CHUNK)
    dst = jnp.concatenate([edge_index[1], pad_dst]).reshape(ROWS, CHUNK)

    xc_lo, xc_hi = _clip(x)
    s1_lo, s1_hi = _seg1(xc_lo, xc_hi, src, dst)
    z = _mid(xc_lo, xc_hi, s1_lo, s1_hi, W1, b1.reshape(1, D), W2)
    z_pair = z.reshape(NPAD // 2, 2 * C)
    s2a, s2b = _seg2(z, src, dst)
    b2p = jnp.concatenate([b2, b2]).reshape(1, 2 * C)
    out = _final(z_pair, s2a.reshape(NPAD // 2, 2 * C),
                 s2b.reshape(NPAD // 2, 2 * C), b2p)
    return out.reshape(NPAD, C)[:N]
